# baseline (speedup 1.0000x reference)
import functools

import jax
import jax.numpy as jnp
from jax import lax
from jax.experimental import pallas as pl
from jax.experimental.pallas import tpu as pltpu
from jax.experimental.pallas import tpu_sc as plsc

D = 128
NC = 2
NS = 16
NW = NC * NS
A = 4096
S = 50
A_PER_W = A // NW
SP = 52
N_CH = A_PER_W // 2
NBUF = 4

_mesh = plsc.VectorSubcoreMesh(core_axis_name="c", subcore_axis_name="s")


@functools.partial(
    pl.kernel,
    out_type=jax.ShapeDtypeStruct((A, S, D), jnp.float32),
    mesh=_mesh,
    scratch_types=[
        pltpu.VMEM((N_CH, 2 * SP), jnp.int32),
        pltpu.VMEM((NBUF, 2 * SP, D), jnp.float32),
        pltpu.SemaphoreType.DMA,
        pltpu.SemaphoreType.DMA,
    ],
)
def _embed(idx_hbm, table_hbm, out_hbm, idx_v, rows_v, gsem, wsem):
    wid = lax.axis_index("s") * NC + lax.axis_index("c")
    a0 = wid * A_PER_W
    pltpu.sync_copy(idx_hbm.at[wid], idx_v)

    def gather(j, b):
        pltpu.async_copy(table_hbm.at[idx_v.at[j]], rows_v.at[b], gsem)

    def wb(j, b):
        pltpu.async_copy(rows_v.at[b, pl.ds(0, S)], out_hbm.at[a0 + 2 * j], wsem)
        pltpu.async_copy(
            rows_v.at[b, pl.ds(SP, S)], out_hbm.at[a0 + 2 * j + 1], wsem
        )

    def wait_gather(b):
        pltpu.make_async_copy(
            table_hbm.at[pl.ds(0, 2 * SP)], rows_v.at[b], gsem
        ).wait()

    def wait_wb_pair():
        pltpu.make_async_copy(
            rows_v.at[0, pl.ds(0, S)], out_hbm.at[a0], wsem
        ).wait()
        pltpu.make_async_copy(
            rows_v.at[0, pl.ds(0, S)], out_hbm.at[a0], wsem
        ).wait()

    for k in range(NBUF - 1):
        gather(k, k)

    @pl.loop(0, N_CH)
    def _(j):
        b = lax.rem(j, NBUF)
        wait_gather(b)
        wb(j, b)
        @pl.when(jnp.logical_and(j > 0, j < N_CH - (NBUF - 1)))
        def _():
            wait_wb_pair()

        @pl.when(j < N_CH - (NBUF - 1))
        def _():
            gather(j + NBUF - 1, lax.rem(j + NBUF - 1, NBUF))

    for _k in range(NBUF):
        wait_wb_pair()


def kernel(x, table):
    idx = jnp.pad(x, ((0, 0), (0, SP - S))).reshape(NW, N_CH, 2 * SP)
    return _embed(idx, table)
